# baseline (speedup 1.0000x reference)
import functools

import jax
import jax.numpy as jnp
from jax.experimental import pallas as pl

N = 10000
E = 320000
DF = 128
DE = 16
H = 256
T = 128

_BLK = 1000


def _combine_body(p_ref, f_ref, x_ref, wc1_ref, bc1_ref, wc2_ref, bc2_ref, o_ref):
    p = p_ref[...]
    f = f_ref[...]
    wc1 = wc1_ref[...]
    hc = (p @ wc1[:T] + f @ wc1[T:]) + bc1_ref[...]
    hc = jnp.maximum(hc, 0.01 * hc)
    o_ref[...] = hc @ wc2_ref[...] + bc2_ref[...] + x_ref[...]


def _combine(past_agg, future_agg, x, Wc1, bc1, Wc2, bc2):
    grid = (N // _BLK,)
    return pl.pallas_call(
        _combine_body,
        grid=grid,
        in_specs=[
            pl.BlockSpec((_BLK, T), lambda i: (i, 0)),
            pl.BlockSpec((_BLK, T), lambda i: (i, 0)),
            pl.BlockSpec((_BLK, DF), lambda i: (i, 0)),
            pl.BlockSpec((2 * T, H), lambda i: (0, 0)),
            pl.BlockSpec((H,), lambda i: (0,)),
            pl.BlockSpec((H, T), lambda i: (0, 0)),
            pl.BlockSpec((T,), lambda i: (0,)),
        ],
        out_specs=pl.BlockSpec((_BLK, T), lambda i: (i, 0)),
        out_shape=jax.ShapeDtypeStruct((N, T), jnp.float32),
    )(past_agg, future_agg, x, Wc1, bc1, Wc2, bc2)


def kernel(x, edge_index, edge_attr, u, batch,
           W1f, b1f, W2f, b2f,
           W1p, b1p, W2p, b2p,
           Wc1, bc1, Wc2, bc2):
    n1 = edge_index[0]
    n2 = edge_index[1]
    Af = x @ W1f[:DF]
    Ap = x @ W1p[:DF]
    Bf = edge_attr @ W1f[DF:] + b1f
    Bp = edge_attr @ W1p[DF:] + b1p
    hf = Af[n2] + Bf
    hf = jnp.maximum(hf, 0.01 * hf)
    hp = Ap[n1] + Bp
    hp = jnp.maximum(hp, 0.01 * hp)
    Gf = jax.ops.segment_sum(hf, n1, num_segments=N)
    Gp = jax.ops.segment_sum(hp, n2, num_segments=N)
    ones = jnp.ones((E,), dtype=jnp.float32)
    cnt1 = jax.ops.segment_sum(ones, n1, num_segments=N)
    cnt2 = jax.ops.segment_sum(ones, n2, num_segments=N)
    future_agg = Gf @ W2f + cnt1[:, None] * b2f
    past_agg = (Gp @ W2p + cnt2[:, None] * b2p) / jnp.maximum(cnt2, 1.0)[:, None]
    return _combine(past_agg, future_agg, x, Wc1, bc1, Wc2, bc2)


# speedup vs baseline: 2.0401x; 2.0401x over previous
import functools

import jax
import jax.numpy as jnp
from jax import lax
from jax.experimental import pallas as pl
from jax.experimental.pallas import tpu as pltpu
from jax.experimental.pallas import tpu_sc as plsc

N = 10000
E = 320000
DF = 128
DE = 16
H = 256
T = 128

HALF = 128
CW = 128
NS = 16
CHUNK = 128
N_CH = E // CHUNK
FULL = N_CH // NS
REM = N_CH % NS
NPAD = 10112
STRIPE = NPAD // NS
PIECES = [(0, 128), (128, 128), (256, 128), (384, 128), (512, 120)]

_NBLK = 1000


def _pre_a_body(x_ref, wf_ref, wp_ref, o_ref):
    xb = x_ref[...]
    wf = wf_ref[...]
    wp = wp_ref[...]
    o_ref[0] = xb @ wf[:, :HALF]
    o_ref[1] = xb @ wf[:, HALF:]
    o_ref[2] = xb @ wp[:, :HALF]
    o_ref[3] = xb @ wp[:, HALF:]


def _pre_a(x, W1fx, W1px):
    return pl.pallas_call(
        _pre_a_body,
        grid=(N // _NBLK,),
        in_specs=[
            pl.BlockSpec((_NBLK, DF), lambda i: (i, 0)),
            pl.BlockSpec((DF, H), lambda i: (0, 0)),
            pl.BlockSpec((DF, H), lambda i: (0, 0)),
        ],
        out_specs=pl.BlockSpec((4, _NBLK, HALF), lambda i: (0, i, 0)),
        out_shape=jax.ShapeDtypeStruct((4, N, HALF), jnp.float32),
    )(x, W1fx, W1px)


_EBLK = 4000


def _pre_b_body(ea_ref, wf_ref, bf_ref, wp_ref, bp_ref, o_ref):
    ea = ea_ref[...]
    wf = wf_ref[...]
    wp = wp_ref[...]
    bf = bf_ref[...]
    bp = bp_ref[...]
    o_ref[0] = ea @ wf[:, :HALF] + bf[:HALF]
    o_ref[1] = ea @ wf[:, HALF:] + bf[HALF:]
    o_ref[2] = ea @ wp[:, :HALF] + bp[:HALF]
    o_ref[3] = ea @ wp[:, HALF:] + bp[HALF:]


def _pre_b(edge_attr, W1fe, b1f, W1pe, b1p):
    return pl.pallas_call(
        _pre_b_body,
        grid=(E // _EBLK,),
        in_specs=[
            pl.BlockSpec((_EBLK, DE), lambda i: (i, 0)),
            pl.BlockSpec((DE, H), lambda i: (0, 0)),
            pl.BlockSpec((H,), lambda i: (0,)),
            pl.BlockSpec((DE, H), lambda i: (0, 0)),
            pl.BlockSpec((H,), lambda i: (0,)),
        ],
        out_specs=pl.BlockSpec((4, _EBLK, HALF), lambda i: (0, i, 0)),
        out_shape=jax.ShapeDtypeStruct((4, E, HALF), jnp.float32),
    )(edge_attr, W1fe, b1f, W1pe, b1p)


def _sc_body(a_hbm, b_hbm, idx_hbm, g_hbm,
             sidx, didx, arow, brow, orow, acc, sem1, sem2, sem3):
    c = lax.axis_index("c")
    s = lax.axis_index("s")
    zero16 = jnp.zeros((16,), jnp.float32)
    base_row = s * STRIPE

    for jj in range(2):
        j = 2 * c + jj

        @pl.loop(0, CHUNK)
        def _(r):
            for q in range(CW // 16):
                orow[r, pl.ds(q * 16, 16)] = zero16

        for off, rows in PIECES:
            pltpu.sync_copy(orow.at[pl.ds(0, rows), :],
                            acc.at[pl.ds(base_row + off, rows), :])

        plsc.subcore_barrier()

        def do_chunk(g):
            base = g * CHUNK
            d_src = pltpu.async_copy(
                idx_hbm.at[pl.ds((1 - c) * E + base, CHUNK)], sidx, sem1)
            d_dst = pltpu.async_copy(
                idx_hbm.at[pl.ds(c * E + base, CHUNK)], didx, sem2)
            d_b = pltpu.async_copy(
                b_hbm.at[pl.ds(j * E + base, CHUNK), :], brow, sem3)
            d_src.wait()

            off = j * N

            @pl.loop(0, CHUNK // 16)
            def _(q):
                sidx[pl.ds(q * 16, 16)] = sidx[pl.ds(q * 16, 16)] + off

            pltpu.async_copy(a_hbm.at[sidx], arow, sem1).wait()
            d_b.wait()

            @pl.loop(0, CHUNK)
            def _(r):
                for q in range(HALF // 16):
                    t = arow[r, pl.ds(q * 16, 16)] + brow[r, pl.ds(q * 16, 16)]
                    orow[r, pl.ds(q * 16, 16)] = jnp.maximum(t, 0.01 * t)

            d_dst.wait()
            pltpu.sync_copy(orow, acc.at[didx], add=True)

        @pl.loop(0, FULL)
        def _(i):
            do_chunk(i * NS + s)

        @pl.when(s < REM)
        def _():
            do_chunk(FULL * NS + s)

        plsc.subcore_barrier()

        for off, rows in PIECES:
            pltpu.sync_copy(acc.at[pl.ds(base_row + off, rows), :],
                            g_hbm.at[j, pl.ds(base_row + off, rows), :])

        if jj == 0:
            plsc.subcore_barrier()


def _sc_gather_scatter(a, b, idx):
    mesh = plsc.VectorSubcoreMesh(core_axis_name="c", subcore_axis_name="s")
    f = pl.kernel(
        _sc_body,
        out_type=jax.ShapeDtypeStruct((4, NPAD, CW), jnp.float32),
        mesh=mesh,
        scratch_types=[
            pltpu.VMEM((CHUNK,), jnp.int32),
            pltpu.VMEM((CHUNK,), jnp.int32),
            pltpu.VMEM((CHUNK, HALF), jnp.float32),
            pltpu.VMEM((CHUNK, HALF), jnp.float32),
            pltpu.VMEM((CHUNK, CW), jnp.float32),
            pltpu.VMEM_SHARED((NPAD, CW), jnp.float32),
            pltpu.SemaphoreType.DMA,
            pltpu.SemaphoreType.DMA,
            pltpu.SemaphoreType.DMA,
        ],
    )
    return f(a, b, idx)


def _post_body(g0, g1, g2, g3, cnt1_ref, cnt2_ref,
               x_ref, w2f_ref, b2f_ref, w2p_ref, b2p_ref,
               wc1_ref, bc1_ref, wc2_ref, bc2_ref, o_ref):
    w2f = w2f_ref[...]
    w2p = w2p_ref[...]
    gf0 = g0[0]
    gf1 = g1[0]
    gp0 = g2[0]
    gp1 = g3[0]
    cnt1 = cnt1_ref[...]
    cnt2 = cnt2_ref[...]
    fa = gf0[:, :HALF] @ w2f[:HALF] + gf1[:, :HALF] @ w2f[HALF:] \
        + cnt1 * b2f_ref[...]
    ps = gp0[:, :HALF] @ w2p[:HALF] + gp1[:, :HALF] @ w2p[HALF:] \
        + cnt2 * b2p_ref[...]
    pa = ps / jnp.maximum(cnt2, 1.0)
    wc1 = wc1_ref[...]
    hc = pa @ wc1[:T] + fa @ wc1[T:] + bc1_ref[...]
    hc = jnp.maximum(hc, 0.01 * hc)
    o_ref[...] = hc @ wc2_ref[...] + bc2_ref[...] + x_ref[...]


def _post(g, cnt1, cnt2, x, W2f, b2f, W2p, b2p, Wc1, bc1, Wc2, bc2):
    gspec = lambda J: pl.BlockSpec((1, _NBLK, CW), lambda i, J=J: (J, i, 0))
    return pl.pallas_call(
        _post_body,
        grid=(N // _NBLK,),
        in_specs=[
            gspec(0), gspec(1), gspec(2), gspec(3),
            pl.BlockSpec((_NBLK, 1), lambda i: (i, 0)),
            pl.BlockSpec((_NBLK, 1), lambda i: (i, 0)),
            pl.BlockSpec((_NBLK, DF), lambda i: (i, 0)),
            pl.BlockSpec((H, T), lambda i: (0, 0)),
            pl.BlockSpec((T,), lambda i: (0,)),
            pl.BlockSpec((H, T), lambda i: (0, 0)),
            pl.BlockSpec((T,), lambda i: (0,)),
            pl.BlockSpec((2 * T, H), lambda i: (0, 0)),
            pl.BlockSpec((H,), lambda i: (0,)),
            pl.BlockSpec((H, T), lambda i: (0, 0)),
            pl.BlockSpec((T,), lambda i: (0,)),
        ],
        out_specs=pl.BlockSpec((_NBLK, T), lambda i: (i, 0)),
        out_shape=jax.ShapeDtypeStruct((N, T), jnp.float32),
    )(g, g, g, g, cnt1, cnt2, x, W2f, b2f, W2p, b2p, Wc1, bc1, Wc2, bc2)


def kernel(x, edge_index, edge_attr, u, batch,
           W1f, b1f, W2f, b2f,
           W1p, b1p, W2p, b2p,
           Wc1, bc1, Wc2, bc2):
    a = _pre_a(x, W1f[:DF], W1p[:DF])
    b = _pre_b(edge_attr, W1f[DF:], b1f, W1p[DF:], b1p)
    idx = edge_index.reshape(2 * E)
    g = _sc_gather_scatter(a.reshape(4 * N, HALF), b.reshape(4 * E, HALF), idx)
    ones = jnp.ones((E,), dtype=jnp.float32)
    cnt1 = jax.ops.segment_sum(ones, edge_index[0], num_segments=N)[:, None]
    cnt2 = jax.ops.segment_sum(ones, edge_index[1], num_segments=N)[:, None]
    return _post(g, cnt1, cnt2, x, W2f, b2f, W2p, b2p, Wc1, bc1, Wc2, bc2)


# speedup vs baseline: 3.3073x; 1.6212x over previous
import dataclasses
import functools

import jax
import jax.numpy as jnp
from jax import lax
from jax.experimental import pallas as pl
from jax.experimental.pallas import tpu as pltpu
from jax.experimental.pallas import tpu_sc as plsc

N = 10000
E = 320000
DF = 128
DE = 16
H = 256
T = 128

HALF = 128
CW = 128
NS = 16
CHUNK = 80
FULL = E // CHUNK // NS
NPAD = 10112
STRIPE = NPAD // NS
PIECES = [(o, 80) for o in range(0, 560, 80)] + [(560, 72)]
NROWS = 80

_NBLK = 1000


def _pre_a_body(x_ref, wf_ref, wp_ref, o_ref):
    xb = x_ref[...]
    wf = wf_ref[...]
    wp = wp_ref[...]
    o_ref[0] = xb @ wf[:, :HALF]
    o_ref[1] = xb @ wf[:, HALF:]
    o_ref[2] = xb @ wp[:, :HALF]
    o_ref[3] = xb @ wp[:, HALF:]


def _pre_a(x, W1fx, W1px):
    return pl.pallas_call(
        _pre_a_body,
        grid=(N // _NBLK,),
        in_specs=[
            pl.BlockSpec((_NBLK, DF), lambda i: (i, 0)),
            pl.BlockSpec((DF, H), lambda i: (0, 0)),
            pl.BlockSpec((DF, H), lambda i: (0, 0)),
        ],
        out_specs=pl.BlockSpec((4, _NBLK, HALF), lambda i: (0, i, 0)),
        out_shape=jax.ShapeDtypeStruct((4, N, HALF), jnp.float32),
    )(x, W1fx, W1px)


_EBLK = 4000


def _pre_b_body(ea_ref, wf_ref, bf_ref, wp_ref, bp_ref, o_ref):
    ea = ea_ref[...]
    wf = wf_ref[...]
    wp = wp_ref[...]
    bf = bf_ref[...]
    bp = bp_ref[...]
    o_ref[0] = ea @ wf[:, :HALF] + bf[:HALF]
    o_ref[1] = ea @ wf[:, HALF:] + bf[HALF:]
    o_ref[2] = ea @ wp[:, :HALF] + bp[:HALF]
    o_ref[3] = ea @ wp[:, HALF:] + bp[HALF:]


def _pre_b(edge_attr, W1fe, b1f, W1pe, b1p):
    return pl.pallas_call(
        _pre_b_body,
        grid=(E // _EBLK,),
        in_specs=[
            pl.BlockSpec((_EBLK, DE), lambda i: (i, 0)),
            pl.BlockSpec((DE, H), lambda i: (0, 0)),
            pl.BlockSpec((H,), lambda i: (0,)),
            pl.BlockSpec((DE, H), lambda i: (0, 0)),
            pl.BlockSpec((H,), lambda i: (0,)),
        ],
        out_specs=pl.BlockSpec((4, _EBLK, HALF), lambda i: (0, i, 0)),
        out_shape=jax.ShapeDtypeStruct((4, E, HALF), jnp.float32),
    )(edge_attr, W1fe, b1f, W1pe, b1p)


def _sc_body(a_hbm, b_hbm, idx_hbm, g_hbm, cnt_hbm,
             sidx, didx, arow, brow, acc, cnt_acc,
             sem_si, sem_di, sem_a, sem_b):
    c = lax.axis_index("c")
    s = lax.axis_index("s")
    zero16 = jnp.zeros((16,), jnp.float32)
    base_row = s * STRIPE

    def zero_arow0():
        @pl.loop(0, CHUNK)
        def _(r):
            for q in range(HALF // 16):
                arow[0][r, pl.ds(q * 16, 16)] = zero16

    def issue_didx(g, b):
        pltpu.async_copy(idx_hbm.at[pl.ds(c * E + g * CHUNK, CHUNK)],
                         didx[b], sem_di[b])

    def wait_didx(b):
        pltpu.make_async_copy(idx_hbm.at[pl.ds(0, CHUNK)],
                              didx[b], sem_di[b]).wait()

    def issue_idx(g, b):
        pltpu.async_copy(idx_hbm.at[pl.ds((1 - c) * E + g * CHUNK, CHUNK)],
                         sidx[b], sem_si[b])
        issue_didx(g, b)

    def wait_sidx(b):
        pltpu.make_async_copy(idx_hbm.at[pl.ds(0, CHUNK)],
                              sidx[b], sem_si[b]).wait()

    def issue_gather(j, g, b):
        off = j * N

        @pl.loop(0, CHUNK // 16)
        def _(q):
            sidx[b][pl.ds(q * 16, 16)] = sidx[b][pl.ds(q * 16, 16)] + off

        pltpu.async_copy(a_hbm.at[sidx[b]], arow[b], sem_a[b])
        pltpu.async_copy(b_hbm.at[pl.ds(j * E + g * CHUNK, CHUNK), :],
                         brow[b], sem_b[b])

    def wait_gather(b):
        pltpu.make_async_copy(a_hbm.at[pl.ds(0, CHUNK), :],
                              arow[b], sem_a[b]).wait()
        pltpu.make_async_copy(b_hbm.at[pl.ds(0, CHUNK), :],
                              brow[b], sem_b[b]).wait()

    def compute_scatter(b):
        @pl.loop(0, CHUNK)
        def _(r):
            for q in range(HALF // 16):
                t = arow[b][r, pl.ds(q * 16, 16)] + brow[b][r, pl.ds(q * 16, 16)]
                arow[b][r, pl.ds(q * 16, 16)] = jnp.maximum(t, 0.01 * t)

        wait_didx(b)
        pltpu.sync_copy(arow[b], acc.at[didx[b]], add=True)

    zero_arow0()

    @pl.when(s == 0)
    def _():
        pltpu.sync_copy(arow[0].at[pl.ds(0, NROWS), :], cnt_acc)

    for q in range(NROWS // 16):
        sidx[1][pl.ds(q * 16, 16)] = lax.iota(jnp.int32, 16) + q * 16

    plsc.subcore_barrier()

    issue_didx(s, 0)

    @pl.loop(0, FULL, step=2)
    def _(i):
        for b in range(2):
            cur = i + b

            @pl.when(cur + 1 < FULL)
            def _():
                issue_didx((cur + 1) * NS + s, 1 - b)

            wait_didx(b)
            for q in range(CHUNK // 16):
                v = didx[b][pl.ds(q * 16, 16)]
                cnts, last = plsc.scan_count(v)
                plsc.addupdate_scatter(
                    arow[0],
                    [jnp.right_shift(v, 7), jnp.bitwise_and(v, 127)],
                    cnts.astype(jnp.float32), mask=last)

    pltpu.sync_copy(arow[0], cnt_acc.at[sidx[1]], add=True)
    plsc.subcore_barrier()

    @pl.when(s == 0)
    def _():
        pltpu.sync_copy(cnt_acc, cnt_hbm.at[c])

    for jj in range(2):
        j = 2 * c + jj

        zero_arow0()
        for off, rows in PIECES:
            pltpu.sync_copy(arow[0].at[pl.ds(0, rows), :],
                            acc.at[pl.ds(base_row + off, rows), :])

        plsc.subcore_barrier()

        def chunk_id(i):
            return i * NS + s

        issue_idx(chunk_id(0), 0)
        issue_idx(chunk_id(1), 1)
        wait_sidx(0)
        issue_gather(j, chunk_id(0), 0)

        @pl.loop(0, FULL, step=2)
        def _(i):
            for b in range(2):
                cur = i + b

                @pl.when(cur + 1 < FULL)
                def _():
                    wait_sidx(1 - b)
                    issue_gather(j, chunk_id(cur + 1), 1 - b)

                wait_gather(b)
                compute_scatter(b)

                @pl.when(cur + 2 < FULL)
                def _():
                    issue_idx(chunk_id(cur + 2), b)

        plsc.subcore_barrier()

        for off, rows in PIECES:
            pltpu.sync_copy(acc.at[pl.ds(base_row + off, rows), :],
                            g_hbm.at[j, pl.ds(base_row + off, rows), :])

        if jj == 0:
            plsc.subcore_barrier()


def _sc_gather_scatter(a, b, idx):
    mesh = plsc.VectorSubcoreMesh(core_axis_name="c", subcore_axis_name="s")
    cp = pltpu.CompilerParams()
    if "needs_layout_passes" in pltpu.CompilerParams.__dataclass_fields__:
        cp = dataclasses.replace(cp, needs_layout_passes=False)
    f = pl.kernel(
        _sc_body,
        compiler_params=cp,
        out_type=(jax.ShapeDtypeStruct((4, NPAD, CW), jnp.float32),
                  jax.ShapeDtypeStruct((2, NROWS, 128), jnp.float32)),
        mesh=mesh,
        scratch_types=[
            [pltpu.VMEM((CHUNK,), jnp.int32) for _ in range(2)],
            [pltpu.VMEM((CHUNK,), jnp.int32) for _ in range(2)],
            [pltpu.VMEM((CHUNK, HALF), jnp.float32) for _ in range(2)],
            [pltpu.VMEM((CHUNK, HALF), jnp.float32) for _ in range(2)],
            pltpu.VMEM_SHARED((NPAD, CW), jnp.float32),
            pltpu.VMEM_SHARED((NROWS, 128), jnp.float32),
            [pltpu.SemaphoreType.DMA for _ in range(2)],
            [pltpu.SemaphoreType.DMA for _ in range(2)],
            [pltpu.SemaphoreType.DMA for _ in range(2)],
            [pltpu.SemaphoreType.DMA for _ in range(2)],
        ],
    )
    return f(a, b, idx)


def _post_body(g0, g1, g2, g3, cnt1_ref, cnt2_ref,
               x_ref, w2f_ref, b2f_ref, w2p_ref, b2p_ref,
               wc1_ref, bc1_ref, wc2_ref, bc2_ref, o_ref):
    w2f = w2f_ref[...]
    w2p = w2p_ref[...]
    gf0 = g0[0]
    gf1 = g1[0]
    gp0 = g2[0]
    gp1 = g3[0]
    cnt1 = cnt1_ref[...]
    cnt2 = cnt2_ref[...]
    fa = gf0[:, :HALF] @ w2f[:HALF] + gf1[:, :HALF] @ w2f[HALF:] \
        + cnt1 * b2f_ref[...]
    ps = gp0[:, :HALF] @ w2p[:HALF] + gp1[:, :HALF] @ w2p[HALF:] \
        + cnt2 * b2p_ref[...]
    pa = ps / jnp.maximum(cnt2, 1.0)
    wc1 = wc1_ref[...]
    hc = pa @ wc1[:T] + fa @ wc1[T:] + bc1_ref[...]
    hc = jnp.maximum(hc, 0.01 * hc)
    o_ref[...] = hc @ wc2_ref[...] + bc2_ref[...] + x_ref[...]


def _post(g, cnt1, cnt2, x, W2f, b2f, W2p, b2p, Wc1, bc1, Wc2, bc2):
    gspec = lambda J: pl.BlockSpec((1, _NBLK, CW), lambda i, J=J: (J, i, 0))
    return pl.pallas_call(
        _post_body,
        grid=(N // _NBLK,),
        in_specs=[
            gspec(0), gspec(1), gspec(2), gspec(3),
            pl.BlockSpec((_NBLK, 1), lambda i: (i, 0)),
            pl.BlockSpec((_NBLK, 1), lambda i: (i, 0)),
            pl.BlockSpec((_NBLK, DF), lambda i: (i, 0)),
            pl.BlockSpec((H, T), lambda i: (0, 0)),
            pl.BlockSpec((T,), lambda i: (0,)),
            pl.BlockSpec((H, T), lambda i: (0, 0)),
            pl.BlockSpec((T,), lambda i: (0,)),
            pl.BlockSpec((2 * T, H), lambda i: (0, 0)),
            pl.BlockSpec((H,), lambda i: (0,)),
            pl.BlockSpec((H, T), lambda i: (0, 0)),
            pl.BlockSpec((T,), lambda i: (0,)),
        ],
        out_specs=pl.BlockSpec((_NBLK, T), lambda i: (i, 0)),
        out_shape=jax.ShapeDtypeStruct((N, T), jnp.float32),
    )(g, g, g, g, cnt1, cnt2, x, W2f, b2f, W2p, b2p, Wc1, bc1, Wc2, bc2)


def kernel(x, edge_index, edge_attr, u, batch,
           W1f, b1f, W2f, b2f,
           W1p, b1p, W2p, b2p,
           Wc1, bc1, Wc2, bc2):
    a = _pre_a(x, W1f[:DF], W1p[:DF])
    b = _pre_b(edge_attr, W1f[DF:], b1f, W1p[DF:], b1p)
    g, cnt = _sc_gather_scatter(
        a.reshape(4 * N, HALF), b.reshape(4 * E, HALF),
        edge_index.reshape(2 * E))
    cnt = cnt.reshape(2, NROWS * 128)
    cnt1 = cnt[0, :N, None]
    cnt2 = cnt[1, :N, None]
    return _post(g, cnt1, cnt2, x, W2f, b2f, W2p, b2p, Wc1, bc1, Wc2, bc2)


# speedup vs baseline: 3.6746x; 1.1111x over previous
import dataclasses
import functools

import jax
import jax.numpy as jnp
from jax import lax
from jax.experimental import pallas as pl
from jax.experimental.pallas import tpu as pltpu
from jax.experimental.pallas import tpu_sc as plsc

N = 10000
E = 320000
DF = 128
DE = 16
H = 256
T = 128

HALF = 128
CW = 128
NS = 16
CHUNK = 80
FULL = E // CHUNK // NS
NPAD = 10112
STRIPE = NPAD // NS
PIECES = [(o, 80) for o in range(0, 560, 80)] + [(560, 72)]
NROWS = 80

_NBLK = 1000


def _pre_a_body(x_ref, wf_ref, wp_ref, o_ref):
    xb = x_ref[...]
    wf = wf_ref[...]
    wp = wp_ref[...]
    o_ref[0] = xb @ wf[:, :HALF]
    o_ref[1] = xb @ wf[:, HALF:]
    o_ref[2] = xb @ wp[:, :HALF]
    o_ref[3] = xb @ wp[:, HALF:]


def _pre_a(x, W1fx, W1px):
    return pl.pallas_call(
        _pre_a_body,
        grid=(N // _NBLK,),
        in_specs=[
            pl.BlockSpec((_NBLK, DF), lambda i: (i, 0)),
            pl.BlockSpec((DF, H), lambda i: (0, 0)),
            pl.BlockSpec((DF, H), lambda i: (0, 0)),
        ],
        out_specs=pl.BlockSpec((4, _NBLK, HALF), lambda i: (0, i, 0)),
        out_shape=jax.ShapeDtypeStruct((4, N, HALF), jnp.float32),
    )(x, W1fx, W1px)


_EBLK = 4000


def _pre_b_body(ea_ref, wf_ref, bf_ref, wp_ref, bp_ref, o_ref):
    ea = ea_ref[...]
    wf = wf_ref[...]
    wp = wp_ref[...]
    bf = bf_ref[...]
    bp = bp_ref[...]
    o_ref[0] = ea @ wf[:, :HALF] + bf[:HALF]
    o_ref[1] = ea @ wf[:, HALF:] + bf[HALF:]
    o_ref[2] = ea @ wp[:, :HALF] + bp[:HALF]
    o_ref[3] = ea @ wp[:, HALF:] + bp[HALF:]


def _pre_b(edge_attr, W1fe, b1f, W1pe, b1p):
    return pl.pallas_call(
        _pre_b_body,
        grid=(E // _EBLK,),
        in_specs=[
            pl.BlockSpec((_EBLK, DE), lambda i: (i, 0)),
            pl.BlockSpec((DE, H), lambda i: (0, 0)),
            pl.BlockSpec((H,), lambda i: (0,)),
            pl.BlockSpec((DE, H), lambda i: (0, 0)),
            pl.BlockSpec((H,), lambda i: (0,)),
        ],
        out_specs=pl.BlockSpec((4, _EBLK, HALF), lambda i: (0, i, 0)),
        out_shape=jax.ShapeDtypeStruct((4, E, HALF), jnp.float32),
    )(edge_attr, W1fe, b1f, W1pe, b1p)


def _sc_body(a_hbm, b_hbm, idx_hbm, g_hbm, cnt_hbm,
             sidx, didx, arow, brow, acc, cnt_acc,
             sem_si, sem_di, sem_a, sem_b, sem_o):
    c = lax.axis_index("c")
    s = lax.axis_index("s")
    zero16 = jnp.zeros((16,), jnp.float32)
    base_row = s * STRIPE

    def zero_brow0():
        @pl.loop(0, CHUNK)
        def _(r):
            for q in range(HALF // 16):
                brow[0][r, pl.ds(q * 16, 16)] = zero16

    def issue_didx(g, b3):
        pltpu.async_copy(idx_hbm.at[pl.ds(c * E + g * CHUNK, CHUNK)],
                         didx[b3], sem_di[b3])

    def wait_didx(b3):
        pltpu.make_async_copy(idx_hbm.at[pl.ds(0, CHUNK)],
                              didx[b3], sem_di[b3]).wait()

    def issue_idx(g, b, b3):
        pltpu.async_copy(idx_hbm.at[pl.ds((1 - c) * E + g * CHUNK, CHUNK)],
                         sidx[b], sem_si[b])
        issue_didx(g, b3)

    def wait_sidx(b):
        pltpu.make_async_copy(idx_hbm.at[pl.ds(0, CHUNK)],
                              sidx[b], sem_si[b]).wait()

    def issue_gather(j, g, b):
        off = j * N

        @pl.loop(0, CHUNK // 16)
        def _(q):
            sidx[b][pl.ds(q * 16, 16)] = sidx[b][pl.ds(q * 16, 16)] + off

        pltpu.async_copy(a_hbm.at[sidx[b]], arow[b], sem_a[b])
        pltpu.async_copy(b_hbm.at[pl.ds(j * E + g * CHUNK, CHUNK), :],
                         brow[b], sem_b[b])

    def wait_gather(b):
        pltpu.make_async_copy(a_hbm.at[pl.ds(0, CHUNK), :],
                              arow[b], sem_a[b]).wait()
        pltpu.make_async_copy(b_hbm.at[pl.ds(0, CHUNK), :],
                              brow[b], sem_b[b]).wait()

    def wait_scatter(b):
        pltpu.make_async_copy(arow[b], acc.at[didx[0]], sem_o[b]).wait()

    def compute(b):
        @pl.loop(0, CHUNK)
        def _(r):
            for q in range(HALF // 16):
                t = arow[b][r, pl.ds(q * 16, 16)] + brow[b][r, pl.ds(q * 16, 16)]
                arow[b][r, pl.ds(q * 16, 16)] = jnp.maximum(t, 0.01 * t)

    def histogram(b3):
        for q in range(CHUNK // 16):
            v = didx[b3][pl.ds(q * 16, 16)]
            cnts, last = plsc.scan_count(v)
            plsc.addupdate_scatter(
                brow[0],
                [jnp.right_shift(v, 7), jnp.bitwise_and(v, 127)],
                cnts.astype(jnp.float32), mask=last)

    zero_brow0()

    @pl.when(s == 0)
    def _():
        pltpu.sync_copy(brow[0].at[pl.ds(0, NROWS), :], cnt_acc)

    for q in range(NROWS // 16):
        sidx[1][pl.ds(q * 16, 16)] = lax.iota(jnp.int32, 16) + q * 16

    plsc.subcore_barrier()

    issue_didx(s, 0)

    @pl.loop(0, FULL, step=2)
    def _(i):
        for b in range(2):
            cur = i + b

            @pl.when(cur + 1 < FULL)
            def _():
                issue_didx((cur + 1) * NS + s, 1 - b)

            wait_didx(b)
            histogram(b)

    pltpu.sync_copy(brow[0], cnt_acc.at[sidx[1]], add=True)
    plsc.subcore_barrier()

    @pl.when(s == 0)
    def _():
        pltpu.sync_copy(cnt_acc, cnt_hbm.at[c])

    for jj in range(2):
        j = 2 * c + jj

        zero_brow0()
        for off, rows in PIECES:
            pltpu.sync_copy(brow[0].at[pl.ds(0, rows), :],
                            acc.at[pl.ds(base_row + off, rows), :])

        plsc.subcore_barrier()

        def chunk_id(i):
            return i * NS + s

        issue_idx(chunk_id(0), 0, 0)
        issue_idx(chunk_id(1), 1, 1)
        wait_sidx(0)
        issue_gather(j, chunk_id(0), 0)

        @pl.loop(0, FULL, step=2)
        def _(i):
            for b in range(2):
                cur = i + b
                b3a = jnp.mod(cur, 3)

                @pl.when(cur >= 1)
                def _():
                    wait_scatter(1 - b)

                @pl.when(cur + 1 < FULL)
                def _():
                    wait_sidx(1 - b)
                    issue_gather(j, chunk_id(cur + 1), 1 - b)

                @pl.when(cur + 2 < FULL)
                def _():
                    for k in range(3):
                        @pl.when(jnp.mod(cur + 2, 3) == k)
                        def _():
                            issue_idx(chunk_id(cur + 2), b, k)

                wait_gather(b)
                compute(b)
                for k in range(3):
                    @pl.when(b3a == k)
                    def _():
                        wait_didx(k)
                        pltpu.async_copy(arow[b], acc.at[didx[k]], sem_o[b],
                                         add=True)

        wait_scatter((FULL - 1) % 2)
        plsc.subcore_barrier()

        for off, rows in PIECES:
            pltpu.sync_copy(acc.at[pl.ds(base_row + off, rows), :],
                            g_hbm.at[j, pl.ds(base_row + off, rows), :])

        if jj == 0:
            plsc.subcore_barrier()


def _sc_gather_scatter(a, b, idx):
    mesh = plsc.VectorSubcoreMesh(core_axis_name="c", subcore_axis_name="s")
    cp = pltpu.CompilerParams()
    if "needs_layout_passes" in pltpu.CompilerParams.__dataclass_fields__:
        cp = dataclasses.replace(cp, needs_layout_passes=False)
    f = pl.kernel(
        _sc_body,
        compiler_params=cp,
        out_type=(jax.ShapeDtypeStruct((4, NPAD, CW), jnp.float32),
                  jax.ShapeDtypeStruct((2, NROWS, 128), jnp.float32)),
        mesh=mesh,
        scratch_types=[
            [pltpu.VMEM((CHUNK,), jnp.int32) for _ in range(2)],
            [pltpu.VMEM((CHUNK,), jnp.int32) for _ in range(3)],
            [pltpu.VMEM((CHUNK, HALF), jnp.float32) for _ in range(2)],
            [pltpu.VMEM((CHUNK, HALF), jnp.float32) for _ in range(2)],
            pltpu.VMEM_SHARED((NPAD, CW), jnp.float32),
            pltpu.VMEM_SHARED((NROWS, 128), jnp.float32),
            [pltpu.SemaphoreType.DMA for _ in range(2)],
            [pltpu.SemaphoreType.DMA for _ in range(3)],
            [pltpu.SemaphoreType.DMA for _ in range(2)],
            [pltpu.SemaphoreType.DMA for _ in range(2)],
            [pltpu.SemaphoreType.DMA for _ in range(2)],
        ],
    )
    return f(a, b, idx)


def _post_body(g0, g1, g2, g3, cnt1_ref, cnt2_ref,
               x_ref, w2f_ref, b2f_ref, w2p_ref, b2p_ref,
               wc1_ref, bc1_ref, wc2_ref, bc2_ref, o_ref):
    w2f = w2f_ref[...]
    w2p = w2p_ref[...]
    gf0 = g0[0]
    gf1 = g1[0]
    gp0 = g2[0]
    gp1 = g3[0]
    cnt1 = cnt1_ref[...]
    cnt2 = cnt2_ref[...]
    fa = gf0[:, :HALF] @ w2f[:HALF] + gf1[:, :HALF] @ w2f[HALF:] \
        + cnt1 * b2f_ref[...]
    ps = gp0[:, :HALF] @ w2p[:HALF] + gp1[:, :HALF] @ w2p[HALF:] \
        + cnt2 * b2p_ref[...]
    pa = ps / jnp.maximum(cnt2, 1.0)
    wc1 = wc1_ref[...]
    hc = pa @ wc1[:T] + fa @ wc1[T:] + bc1_ref[...]
    hc = jnp.maximum(hc, 0.01 * hc)
    o_ref[...] = hc @ wc2_ref[...] + bc2_ref[...] + x_ref[...]


def _post(g, cnt1, cnt2, x, W2f, b2f, W2p, b2p, Wc1, bc1, Wc2, bc2):
    gspec = lambda J: pl.BlockSpec((1, _NBLK, CW), lambda i, J=J: (J, i, 0))
    return pl.pallas_call(
        _post_body,
        grid=(N // _NBLK,),
        in_specs=[
            gspec(0), gspec(1), gspec(2), gspec(3),
            pl.BlockSpec((_NBLK, 1), lambda i: (i, 0)),
            pl.BlockSpec((_NBLK, 1), lambda i: (i, 0)),
            pl.BlockSpec((_NBLK, DF), lambda i: (i, 0)),
            pl.BlockSpec((H, T), lambda i: (0, 0)),
            pl.BlockSpec((T,), lambda i: (0,)),
            pl.BlockSpec((H, T), lambda i: (0, 0)),
            pl.BlockSpec((T,), lambda i: (0,)),
            pl.BlockSpec((2 * T, H), lambda i: (0, 0)),
            pl.BlockSpec((H,), lambda i: (0,)),
            pl.BlockSpec((H, T), lambda i: (0, 0)),
            pl.BlockSpec((T,), lambda i: (0,)),
        ],
        out_specs=pl.BlockSpec((_NBLK, T), lambda i: (i, 0)),
        out_shape=jax.ShapeDtypeStruct((N, T), jnp.float32),
    )(g, g, g, g, cnt1, cnt2, x, W2f, b2f, W2p, b2p, Wc1, bc1, Wc2, bc2)


def kernel(x, edge_index, edge_attr, u, batch,
           W1f, b1f, W2f, b2f,
           W1p, b1p, W2p, b2p,
           Wc1, bc1, Wc2, bc2):
    a = _pre_a(x, W1f[:DF], W1p[:DF])
    b = _pre_b(edge_attr, W1f[DF:], b1f, W1p[DF:], b1p)
    g, cnt = _sc_gather_scatter(
        a.reshape(4 * N, HALF), b.reshape(4 * E, HALF),
        edge_index.reshape(2 * E))
    cnt = cnt.reshape(2, NROWS * 128)
    cnt1 = cnt[0, :N, None]
    cnt2 = cnt[1, :N, None]
    return _post(g, cnt1, cnt2, x, W2f, b2f, W2p, b2p, Wc1, bc1, Wc2, bc2)
